# baseline (speedup 1.0000x reference)
import jax
import jax.numpy as jnp
from jax.experimental import pallas as pl
from jax.experimental.pallas import tpu as pltpu

_B, _N, _IN, _HID, _NH = 4, 1024, 256, 256, 8
_DH = _HID // _NH
_LOG2E = 1.4426950408889634


def _gat_kernel(x_ref, adj_ref, w_ref, asrc_ref, adst_ref, out_ref):
    xb = x_ref[0]
    h = jnp.dot(xb, w_ref[...], preferred_element_type=jnp.float32)
    es = jnp.dot(h, asrc_ref[...], preferred_element_type=jnp.float32)
    ed = jnp.dot(h, adst_ref[...], preferred_element_type=jnp.float32)
    m_ub = jnp.maximum(es + jnp.max(ed, axis=0, keepdims=True), 0.0)
    a1 = ((es - m_ub) * _LOG2E).astype(jnp.bfloat16)
    a2 = ((es * 0.2 - m_ub) * _LOG2E).astype(jnp.bfloat16)
    edt = ed.T
    b1 = (edt * _LOG2E).astype(jnp.bfloat16)
    b2 = (edt * (0.2 * _LOG2E)).astype(jnp.bfloat16)
    adj_bf = adj_ref[0].astype(jnp.bfloat16)
    hb = h.astype(jnp.bfloat16)
    h_aug = jnp.concatenate(
        [hb.reshape(_N, _NH, _DH),
         jnp.ones((_N, _NH, _DH), dtype=jnp.bfloat16)],
        axis=2).reshape(_N, _NH * 2 * _DH)
    h_mean = jnp.mean(h, axis=0, keepdims=True)
    outs = []
    for i in range(_NH):
        v1 = a1[:, i:i + 1] + b1[i:i + 1, :]
        v2 = a2[:, i:i + 1] + b2[i:i + 1, :]
        p = jnp.exp2(jnp.maximum(v1, v2)) * adj_bf
        r = jnp.dot(p, h_aug[:, 2 * _DH * i:2 * _DH * (i + 1)],
                    preferred_element_type=jnp.float32)
        o, s = r[:, :_DH], r[:, _DH:2 * _DH]
        outs.append(jnp.where(s > 0, o / s, h_mean[:, i * _DH:(i + 1) * _DH]))
    hcat = jnp.concatenate(outs, axis=1)
    out_ref[0] = jnp.maximum(hcat, 0.0) + xb


def _build_call(interpret=False):
    grid = (_B,)
    return pl.pallas_call(
        _gat_kernel,
        grid=grid,
        in_specs=[
            pl.BlockSpec((1, _N, _IN), lambda b: (b, 0, 0)),
            pl.BlockSpec((1, _N, _N), lambda b: (b, 0, 0)),
            pl.BlockSpec((_IN, _HID), lambda b: (0, 0)),
            pl.BlockSpec((_HID, _NH), lambda b: (0, 0)),
            pl.BlockSpec((_HID, _NH), lambda b: (0, 0)),
        ],
        out_specs=pl.BlockSpec((1, _N, _HID), lambda b: (b, 0, 0)),
        out_shape=jax.ShapeDtypeStruct((_B, _N, _HID), jnp.float32),
        compiler_params=pltpu.CompilerParams(
            dimension_semantics=("parallel",),
        ),
        interpret=interpret,
    )


def kernel(x, adj, W, a_src, a_dst):
    Wfull = jnp.transpose(W, (1, 0, 2)).reshape(_IN, _HID)
    eye = jnp.eye(_NH, dtype=jnp.float32)
    Asrc = (a_src[:, :, None] * eye[:, None, :]).reshape(_HID, _NH)
    Adst = (a_dst[:, :, None] * eye[:, None, :]).reshape(_HID, _NH)
    return _build_call()(x, adj, Wfull, Asrc, Adst)


# speedup vs baseline: 1.0222x; 1.0222x over previous
import jax
import jax.numpy as jnp
from jax.experimental import pallas as pl
from jax.experimental.pallas import tpu as pltpu

_B, _N, _IN, _HID, _NH = 4, 1024, 256, 256, 8
_DH = _HID // _NH
_LOG2E = 1.4426950408889634


def _gat_kernel(x_ref, adj_ref, w_ref, asrc_ref, adst_ref, out_ref):
    xb = x_ref[0]
    h = jnp.dot(xb, w_ref[...], preferred_element_type=jnp.float32)
    es = jnp.dot(h, asrc_ref[...], preferred_element_type=jnp.float32)
    ed = jnp.dot(h, adst_ref[...], preferred_element_type=jnp.float32)
    m_ub = jnp.maximum(es + jnp.max(ed, axis=0, keepdims=True), 0.0)
    a1 = ((es - m_ub) * _LOG2E).astype(jnp.bfloat16)
    a2 = ((es * 0.2 - m_ub) * _LOG2E).astype(jnp.bfloat16)
    edt = ed.T
    b1 = (edt * _LOG2E).astype(jnp.bfloat16)
    b2 = (edt * (0.2 * _LOG2E)).astype(jnp.bfloat16)
    adj_bf = adj_ref[0].astype(jnp.bfloat16)
    hb = h.astype(jnp.bfloat16)
    ones_col = jnp.ones((_N, 1), dtype=jnp.bfloat16)
    h_mean = jnp.mean(h, axis=0, keepdims=True)
    outs, dens = [], []
    for i in range(_NH):
        v1 = a1[:, i:i + 1] + b1[i:i + 1, :]
        v2 = a2[:, i:i + 1] + b2[i:i + 1, :]
        p = jnp.exp2(jnp.maximum(v1, v2)) * adj_bf
        h_aug = jnp.concatenate([hb[:, i * _DH:(i + 1) * _DH], ones_col], axis=1)
        r = jnp.dot(p, h_aug, preferred_element_type=jnp.float32)
        outs.append(r[:, :_DH])
        dens.append(r[:, _DH:_DH + 1])
    o_cat = jnp.concatenate(outs, axis=1)
    s8 = jnp.concatenate(dens, axis=1)
    s_rep = jnp.broadcast_to(s8[:, :, None], (_N, _NH, _DH)).reshape(_N, _HID)
    hm_rep = jnp.broadcast_to(h_mean, (_N, _HID))
    hcat = jnp.where(s_rep > 0, o_cat / s_rep, hm_rep)
    out_ref[0] = jnp.maximum(hcat, 0.0) + xb


def _build_call(interpret=False):
    grid = (_B,)
    return pl.pallas_call(
        _gat_kernel,
        grid=grid,
        in_specs=[
            pl.BlockSpec((1, _N, _IN), lambda b: (b, 0, 0)),
            pl.BlockSpec((1, _N, _N), lambda b: (b, 0, 0)),
            pl.BlockSpec((_IN, _HID), lambda b: (0, 0)),
            pl.BlockSpec((_HID, _NH), lambda b: (0, 0)),
            pl.BlockSpec((_HID, _NH), lambda b: (0, 0)),
        ],
        out_specs=pl.BlockSpec((1, _N, _HID), lambda b: (b, 0, 0)),
        out_shape=jax.ShapeDtypeStruct((_B, _N, _HID), jnp.float32),
        compiler_params=pltpu.CompilerParams(
            dimension_semantics=("parallel",),
        ),
        interpret=interpret,
    )


def kernel(x, adj, W, a_src, a_dst):
    Wfull = jnp.transpose(W, (1, 0, 2)).reshape(_IN, _HID)
    eye = jnp.eye(_NH, dtype=jnp.float32)
    Asrc = (a_src[:, :, None] * eye[:, None, :]).reshape(_HID, _NH)
    Adst = (a_dst[:, :, None] * eye[:, None, :]).reshape(_HID, _NH)
    return _build_call()(x, adj, Wfull, Asrc, Adst)


# speedup vs baseline: 1.1547x; 1.1296x over previous
import jax
import jax.numpy as jnp
from jax.experimental import pallas as pl
from jax.experimental.pallas import tpu as pltpu

_B, _N, _IN, _HID, _NH = 4, 1024, 256, 256, 8
_DH = _HID // _NH
_LOG2E = 1.4426950408889634


def _gat_kernel(x_ref, adj_ref, w_ref, asrc_ref, adst_ref, out_ref):
    xb = x_ref[0]
    h = jnp.dot(xb, w_ref[...], preferred_element_type=jnp.float32)
    es = jnp.dot(h, asrc_ref[...], preferred_element_type=jnp.float32)
    ed = jnp.dot(h, adst_ref[...], preferred_element_type=jnp.float32)
    m_ub = jnp.maximum(es + jnp.max(ed, axis=0, keepdims=True), 0.0)
    a1 = ((es - m_ub) * _LOG2E).astype(jnp.bfloat16)
    a2 = ((es * 0.2 - m_ub) * _LOG2E).astype(jnp.bfloat16)
    edt = ed.T
    b1 = (edt * _LOG2E).astype(jnp.bfloat16)
    b2 = (edt * (0.2 * _LOG2E)).astype(jnp.bfloat16)
    adj_bf = adj_ref[0].astype(jnp.bfloat16)
    hb = h.astype(jnp.bfloat16)
    ones_col = jnp.ones((_N, 1), dtype=jnp.bfloat16)
    h_mean = jnp.mean(h, axis=0, keepdims=True)
    outs = []
    for i in range(_NH):
        v1 = a1[:, i:i + 1] + b1[i:i + 1, :]
        v2 = a2[:, i:i + 1] + b2[i:i + 1, :]
        p = jnp.exp2(jnp.maximum(v1, v2)) * adj_bf
        h_aug = jnp.concatenate([hb[:, i * _DH:(i + 1) * _DH], ones_col], axis=1)
        r = jnp.dot(p, h_aug, preferred_element_type=jnp.float32)
        o, s = r[:, :_DH], r[:, _DH:_DH + 1]
        outs.append(jnp.where(s > 0, o / s, h_mean[:, i * _DH:(i + 1) * _DH]))
    hcat = jnp.concatenate(outs, axis=1)
    out_ref[0] = jnp.maximum(hcat, 0.0) + xb


def _build_call(interpret=False):
    grid = (_B,)
    return pl.pallas_call(
        _gat_kernel,
        grid=grid,
        in_specs=[
            pl.BlockSpec((1, _N, _IN), lambda b: (b, 0, 0)),
            pl.BlockSpec((1, _N, _N), lambda b: (b, 0, 0)),
            pl.BlockSpec((_IN, _HID), lambda b: (0, 0)),
            pl.BlockSpec((_HID, _NH), lambda b: (0, 0)),
            pl.BlockSpec((_HID, _NH), lambda b: (0, 0)),
        ],
        out_specs=pl.BlockSpec((1, _N, _HID), lambda b: (b, 0, 0)),
        out_shape=jax.ShapeDtypeStruct((_B, _N, _HID), jnp.float32),
        compiler_params=pltpu.CompilerParams(
            dimension_semantics=("parallel",),
        ),
        interpret=interpret,
    )


def kernel(x, adj, W, a_src, a_dst):
    Wfull = jnp.transpose(W, (1, 0, 2)).reshape(_IN, _HID)
    eye = jnp.eye(_NH, dtype=jnp.float32)
    Asrc = (a_src[:, :, None] * eye[:, None, :]).reshape(_HID, _NH)
    Adst = (a_dst[:, :, None] * eye[:, None, :]).reshape(_HID, _NH)
    return _build_call()(x, adj, Wfull, Asrc, Adst)
